# baseline (speedup 1.0000x reference)
import functools

import jax
import jax.numpy as jnp
from jax import lax
from jax.experimental import pallas as pl
from jax.experimental.pallas import tpu as pltpu
from jax.experimental.pallas import tpu_sc as plsc

VOCAB = 100000
DIM = 128
BATCH = 4096
SEQ = 50

_INFO = plsc.get_sparse_core_info()
NC = _INFO.num_cores
NS = _INFO.num_subcores
NW = NC * NS

BPW = BATCH // NW
LPG = 2
NSLOT = 2 * SEQ // LPG
NBUF = 2


@functools.partial(
    pl.kernel,
    out_type=jax.ShapeDtypeStruct((SEQ, BATCH, 2 * DIM), jnp.float32),
    mesh=plsc.VectorSubcoreMesh(core_axis_name="c", subcore_axis_name="s"),
    scratch_types=[
        pltpu.VMEM(((SEQ // LPG) * LPG * BPW,), jnp.int32),
        pltpu.VMEM(((SEQ // LPG) * LPG * BPW,), jnp.int32),
        pltpu.VMEM((NBUF, LPG, BPW, DIM), jnp.float32),
        pltpu.SemaphoreType.DMA,
        pltpu.SemaphoreType.DMA,
    ],
)
def _gather_rows(firsts_hbm, lasts_hbm, table_hbm, out_hbm,
                 firsts_v, lasts_v, rows_v, gsem, ssem):
    wid = lax.axis_index("s") * NC + lax.axis_index("c")
    b0 = wid * BPW
    pltpu.sync_copy(firsts_hbm.at[wid], firsts_v)
    pltpu.sync_copy(lasts_hbm.at[wid], lasts_v)

    def fire_gather(s, h, buf):
        idx = (firsts_v if h == 0 else lasts_v).at[
            pl.ds(lax.div(s, 2) * (LPG * BPW), LPG * BPW)]
        pltpu.async_copy(table_hbm.at[idx],
                         rows_v.at[buf].reshape(LPG * BPW, DIM), gsem)

    def wait_gather(s, h, buf):
        idx = (firsts_v if h == 0 else lasts_v).at[
            pl.ds(lax.div(s, 2) * (LPG * BPW), LPG * BPW)]
        pltpu.make_async_copy(table_hbm.at[idx],
                              rows_v.at[buf].reshape(LPG * BPW, DIM),
                              gsem).wait()

    fire_gather(0, 0, 0)

    @pl.loop(0, NSLOT, step=NBUF)
    def _body(s0):
        for k in range(NBUF):
            s = s0 + k
            h = k % 2
            dst = out_hbm.at[pl.ds(LPG * lax.div(s, 2), LPG),
                             pl.ds(b0, BPW), pl.ds(h * DIM, DIM)]
            wait_gather(s, h, k)
            pltpu.async_copy(rows_v.at[k], dst, ssem)

            @pl.when(s >= 1)
            def _():
                pltpu.make_async_copy(rows_v.at[k], dst, ssem).wait()

            @pl.when(s + 1 < NSLOT)
            def _():
                fire_gather(s + 1, 1 - h, (k + 1) % NBUF)

    dst0 = out_hbm.at[pl.ds(0, LPG), pl.ds(b0, BPW), pl.ds(0, DIM)]
    pltpu.make_async_copy(rows_v.at[0], dst0, ssem).wait()


def _arrange(plane):
    t = plane.reshape(SEQ // LPG, LPG, NW, BPW)
    return t.transpose(2, 0, 1, 3).reshape(NW, (SEQ // LPG) * LPG * BPW)


def kernel(first_last_ids, table):
    ids = first_last_ids.astype(jnp.int32)
    firsts_t = jnp.transpose(ids[..., 0])
    lasts_t = jnp.transpose(ids[..., 1])
    out = _gather_rows(_arrange(firsts_t), _arrange(lasts_t), table)
    return jnp.transpose(out, (1, 0, 2))


# speedup vs baseline: 1.0641x; 1.0641x over previous
import functools

import jax
import jax.numpy as jnp
from jax import lax
from jax.experimental import pallas as pl
from jax.experimental.pallas import tpu as pltpu
from jax.experimental.pallas import tpu_sc as plsc

VOCAB = 100000
DIM = 128
BATCH = 4096
SEQ = 50

_INFO = plsc.get_sparse_core_info()
NC = _INFO.num_cores
NS = _INFO.num_subcores
NW = NC * NS

BPW = BATCH // NW
NSLOT = 2 * SEQ
NBUF = 6
DEPTH = 5


@functools.partial(
    pl.kernel,
    out_type=jax.ShapeDtypeStruct((SEQ, BATCH, 2 * DIM), jnp.float32),
    mesh=plsc.VectorSubcoreMesh(core_axis_name="c", subcore_axis_name="s"),
    scratch_types=[
        pltpu.VMEM((SEQ, BPW), jnp.int32),
        pltpu.VMEM((SEQ, BPW), jnp.int32),
        pltpu.VMEM((NBUF, BPW, DIM), jnp.float32),
        pltpu.SemaphoreType.DMA,
        pltpu.SemaphoreType.DMA,
    ],
)
def _gather_rows(firsts_hbm, lasts_hbm, table_hbm, out_hbm,
                 firsts_v, lasts_v, rows_v, gsem, ssem):
    wid = lax.axis_index("s") * NC + lax.axis_index("c")
    b0 = wid * BPW
    pltpu.sync_copy(firsts_hbm.at[:, pl.ds(b0, BPW)], firsts_v)
    pltpu.sync_copy(lasts_hbm.at[:, pl.ds(b0, BPW)], lasts_v)

    def fire_gather(s, h, buf):
        idx = (firsts_v if h == 0 else lasts_v).at[lax.div(s, 2)]
        pltpu.async_copy(table_hbm.at[idx], rows_v.at[buf], gsem)

    def wait_gather(s, h, buf):
        idx = (firsts_v if h == 0 else lasts_v).at[lax.div(s, 2)]
        pltpu.make_async_copy(table_hbm.at[idx], rows_v.at[buf], gsem).wait()

    for p in range(DEPTH):
        fire_gather(p, p % 2, p)

    @pl.loop(0, NSLOT, step=2)
    def _body(s0):
        for k in range(2):
            s = s0 + k
            h = k
            buf = lax.rem(s, NBUF)
            dst = out_hbm.at[lax.div(s, 2), pl.ds(b0, BPW),
                             pl.ds(h * DIM, DIM)]
            wait_gather(s, h, buf)
            pltpu.async_copy(rows_v.at[buf], dst, ssem)

            @pl.when(s >= 1)
            def _():
                pltpu.make_async_copy(rows_v.at[buf], dst, ssem).wait()

            @pl.when(s + DEPTH < NSLOT)
            def _():
                fire_gather(s + DEPTH, (k + DEPTH) % 2,
                            lax.rem(s + DEPTH, NBUF))

    dst0 = out_hbm.at[0, pl.ds(b0, BPW), pl.ds(0, DIM)]
    pltpu.make_async_copy(rows_v.at[0], dst0, ssem).wait()


def kernel(first_last_ids, table):
    ids = first_last_ids.astype(jnp.int32)
    firsts_t = jnp.transpose(ids[..., 0])
    lasts_t = jnp.transpose(ids[..., 1])
    out = _gather_rows(firsts_t, lasts_t, table)
    return jnp.transpose(out, (1, 0, 2))
